# baseline (speedup 1.0000x reference)
import jax
import jax.numpy as jnp
from jax.experimental import pallas as pl
from jax.experimental.pallas import tpu as pltpu


def _moe_body(n_ref, x_ref, W1_ref, b1_ref, W2_ref, b2_ref, W3_ref, b3_ref,
              Wg_ref, bg_ref, loss_ref, idx_ref,
              featb, glog, mse_s):
    B = x_ref.shape[0]
    IN = W1_ref.shape[2]
    E = W1_ref.shape[1]
    K = n_ref.shape[1]
    D = 128
    TK = IN // D
    DT = TK // K
    NJ = pl.num_programs(0)
    j = pl.program_id(0)
    f32 = jnp.float32

    def xsl(t, n):
        s = pl.multiple_of((t * NJ + n) * D, D)
        return x_ref[:, pl.ds(s, D)]

    for t in range(DT):
        for k in range(K):
            featb[:, (t * K + k) * D:(t * K + k) * D + D] = (
                xsl(t, n_ref[j, k]).astype(jnp.bfloat16))

    wg = Wg_ref[0].astype(jnp.bfloat16)
    glog[...] = (jnp.dot(featb[...], wg, preferred_element_type=f32)
                 + bg_ref[0])

    tgt = xsl(DT // 2, j)
    a = featb[...]

    for u in range(E):
        w1 = W1_ref[0, u].astype(jnp.bfloat16)
        h = jnp.dot(a, w1, preferred_element_type=f32) + b1_ref[0, u]
        h = jnp.maximum(h, 0.0)
        w2 = W2_ref[0, u].astype(jnp.bfloat16)
        h2 = jnp.dot(h.astype(jnp.bfloat16), w2, preferred_element_type=f32) + b2_ref[0, u]
        h2 = jnp.maximum(h2, 0.0)
        w3 = W3_ref[0, u].astype(jnp.bfloat16)
        pred = jnp.dot(h2.astype(jnp.bfloat16), w3, preferred_element_type=f32) + b3_ref[0, u]
        d = pred - tgt
        mse_s[u] = jnp.mean(d * d, axis=1)[None, :]

    mm = mse_s[:, 0, :]
    gT = jnp.transpose(glog[...])
    gT = gT - jnp.max(gT, axis=0, keepdims=True)
    p = jnp.exp(gT)
    gate = p / jnp.sum(p, axis=0, keepdims=True)
    contrib = jnp.sum(gate * mm) / (B * NJ)

    @pl.when(j == 0)
    def _():
        loss_ref[0, 0] = contrib

    @pl.when(j > 0)
    def _():
        loss_ref[0, 0] += contrib

    @pl.when(j == NJ - 1)
    def _argmin_last_joint():
        bv = mm[0:1, :]
        bi = jnp.zeros((1, B), jnp.int32)
        for u in range(1, E):
            ru = mm[u:u + 1, :]
            take = ru < bv
            bi = jnp.where(take, u, bi)
            bv = jnp.where(take, ru, bv)
        idx_ref[...] = bi


def kernel(x, W1, b1, W2, b2, W3, b3, Wg, bg, neighbors):
    B, DT, J, D = x.shape
    _, E, IN, H = W1.shape
    DOUT = W3.shape[-1]

    x2 = x.reshape(B, DT * J * D)
    b1r = b1.reshape(J, E, 1, H)
    b2r = b2.reshape(J, E, 1, H)
    b3r = b3.reshape(J, E, 1, DOUT)
    bgr = bg.reshape(J, 1, E)

    def c00(j):
        return (0, 0)

    loss2d, idx2d = pl.pallas_call(
        _moe_body,
        grid=(J,),
        in_specs=[
            pl.BlockSpec(memory_space=pltpu.SMEM),
            pl.BlockSpec((B, DT * J * D), lambda j: (0, 0)),
            pl.BlockSpec((1, E, IN, H), lambda j: (j, 0, 0, 0)),
            pl.BlockSpec((1, E, 1, H), lambda j: (j, 0, 0, 0)),
            pl.BlockSpec((1, E, H, H), lambda j: (j, 0, 0, 0)),
            pl.BlockSpec((1, E, 1, H), lambda j: (j, 0, 0, 0)),
            pl.BlockSpec((1, E, H, DOUT), lambda j: (j, 0, 0, 0)),
            pl.BlockSpec((1, E, 1, DOUT), lambda j: (j, 0, 0, 0)),
            pl.BlockSpec((1, IN, E), lambda j: (j, 0, 0)),
            pl.BlockSpec((1, 1, E), lambda j: (j, 0, 0)),
        ],
        out_specs=[
            pl.BlockSpec((1, 1), c00, memory_space=pltpu.SMEM),
            pl.BlockSpec((1, B), c00),
        ],
        out_shape=[
            jax.ShapeDtypeStruct((1, 1), jnp.float32),
            jax.ShapeDtypeStruct((1, B), jnp.int32),
        ],
        scratch_shapes=[
            pltpu.VMEM((B, IN), jnp.bfloat16),
            pltpu.VMEM((B, E), jnp.float32),
            pltpu.VMEM((E, 1, B), jnp.float32),
        ],
    )(neighbors, x2, W1, b1r, W2, b2r, W3, b3r, Wg, bgr)

    return loss2d[0, 0], idx2d[0]


# speedup vs baseline: 1.0101x; 1.0101x over previous
import jax
import jax.numpy as jnp
from jax.experimental import pallas as pl
from jax.experimental.pallas import tpu as pltpu


def _moe_body(n_ref, x_ref, W1_ref, b1_ref, W2_ref, b2_ref, W3_ref, b3_ref,
              Wg_ref, bg_ref, loss_ref, idx_ref,
              featb, glog, mse_s):
    B = x_ref.shape[0]
    IN = W1_ref.shape[2]
    E = W1_ref.shape[1]
    K = n_ref.shape[1]
    D = 128
    TK = IN // D
    DT = TK // K
    NJ = pl.num_programs(0)
    j = pl.program_id(0)
    f32 = jnp.float32

    def xsl(t, n):
        s = pl.multiple_of((t * NJ + n) * D, D)
        return x_ref[:, pl.ds(s, D)]

    for t in range(DT):
        for k in range(K):
            featb[:, (t * K + k) * D:(t * K + k) * D + D] = (
                xsl(t, n_ref[j, k]).astype(jnp.bfloat16))

    wg = Wg_ref[0].astype(jnp.bfloat16)
    glog[...] = (jnp.dot(featb[...], wg, preferred_element_type=f32)
                 + bg_ref[0])

    tgt = xsl(DT // 2, j)
    a = featb[...]

    for u in range(E):
        w1 = W1_ref[0, u].astype(jnp.bfloat16)
        h = jnp.dot(a, w1, preferred_element_type=f32) + b1_ref[0, u]
        h = jnp.maximum(h, 0.0)
        w2 = W2_ref[0, u].astype(jnp.bfloat16)
        h2 = jnp.dot(h.astype(jnp.bfloat16), w2, preferred_element_type=f32) + b2_ref[0, u]
        h2 = jnp.maximum(h2, 0.0)
        w3 = W3_ref[0, u].astype(jnp.bfloat16)
        pred = jnp.dot(h2.astype(jnp.bfloat16), w3, preferred_element_type=f32) + b3_ref[0, u]
        d = pred - tgt
        mse_s[:, u:u + 1] = jnp.mean(d * d, axis=1, keepdims=True)

    mm = mse_s[...]
    g = glog[...]
    g = g - jnp.max(g, axis=1, keepdims=True)
    p = jnp.exp(g)
    gate = p / jnp.sum(p, axis=1, keepdims=True)
    contrib = jnp.sum(gate * mm) / (B * NJ)

    @pl.when(j == 0)
    def _():
        loss_ref[0, 0] = contrib

    @pl.when(j > 0)
    def _():
        loss_ref[0, 0] += contrib

    @pl.when(j == NJ - 1)
    def _argmin_last_joint():
        bv = mm[:, 0:1]
        bi = jnp.zeros((B, 1), jnp.int32)
        for u in range(1, E):
            ru = mm[:, u:u + 1]
            take = ru < bv
            bi = jnp.where(take, u, bi)
            bv = jnp.where(take, ru, bv)
        idx_ref[...] = bi


def kernel(x, W1, b1, W2, b2, W3, b3, Wg, bg, neighbors):
    B, DT, J, D = x.shape
    _, E, IN, H = W1.shape
    DOUT = W3.shape[-1]

    x2 = x.reshape(B, DT * J * D)
    b1r = b1.reshape(J, E, 1, H)
    b2r = b2.reshape(J, E, 1, H)
    b3r = b3.reshape(J, E, 1, DOUT)
    bgr = bg.reshape(J, 1, E)

    def c00(j):
        return (0, 0)

    loss2d, idx2d = pl.pallas_call(
        _moe_body,
        grid=(J,),
        in_specs=[
            pl.BlockSpec(memory_space=pltpu.SMEM),
            pl.BlockSpec((B, DT * J * D), lambda j: (0, 0)),
            pl.BlockSpec((1, E, IN, H), lambda j: (j, 0, 0, 0)),
            pl.BlockSpec((1, E, 1, H), lambda j: (j, 0, 0, 0)),
            pl.BlockSpec((1, E, H, H), lambda j: (j, 0, 0, 0)),
            pl.BlockSpec((1, E, 1, H), lambda j: (j, 0, 0, 0)),
            pl.BlockSpec((1, E, H, DOUT), lambda j: (j, 0, 0, 0)),
            pl.BlockSpec((1, E, 1, DOUT), lambda j: (j, 0, 0, 0)),
            pl.BlockSpec((1, IN, E), lambda j: (j, 0, 0)),
            pl.BlockSpec((1, 1, E), lambda j: (j, 0, 0)),
        ],
        out_specs=[
            pl.BlockSpec((1, 1), c00, memory_space=pltpu.SMEM),
            pl.BlockSpec((B, 1), c00),
        ],
        out_shape=[
            jax.ShapeDtypeStruct((1, 1), jnp.float32),
            jax.ShapeDtypeStruct((B, 1), jnp.int32),
        ],
        scratch_shapes=[
            pltpu.VMEM((B, IN), jnp.bfloat16),
            pltpu.VMEM((B, E), jnp.float32),
            pltpu.VMEM((B, E), jnp.float32),
        ],
    )(neighbors, x2, W1, b1r, W2, b2r, W3, b3r, Wg, bgr)

    return loss2d[0, 0], idx2d[:, 0]
